# Initial kernel scaffold; baseline (speedup 1.0000x reference)
import jax
import jax.numpy as jnp
from jax.experimental import pallas as pl


def kernel(x, edge_index, batch, W_in, b_in, ln_in_g, ln_in_b, W1, b1, n1_g, n1_b, W2, b2, n2_g, n2_b, W3, b3, n3_g, n3_b, Wo, bo):
    raise NotImplementedError("write your pallas kernel here")


# speedup vs baseline: 11.5857x; 11.5857x over previous
import functools

import jax
import jax.numpy as jnp
from jax import lax
from jax.experimental import pallas as pl
from jax.experimental.pallas import tpu as pltpu
from jax.experimental.pallas import tpu_sc as plsc

_N = 10000
_E = 320000
_NG = 16
_TILES = 16
_CORES = 2
_K = 125
_NP = 10240
_RPT = _NP // _TILES
_BLK = 1000
_GRID = _N // _BLK

_F32 = jnp.float32
_HI = lax.Precision.HIGHEST


def _dot(a, b):
    return lax.dot_general(a, b, (((a.ndim - 1,), (0,)), ((), ())),
                           precision=_HI, preferred_element_type=_F32)


def _ln(t, g, b, eps=1e-5):
    mu = jnp.mean(t, axis=-1, keepdims=True)
    var = jnp.mean((t - mu) ** 2, axis=-1, keepdims=True)
    return (t - mu) * lax.rsqrt(var + eps) * g + b


def _dinv_of(deg_ref):
    return lax.rsqrt(deg_ref[0, :, 0:1] + deg_ref[1, :, 0:1] + 1.0)


def _sc_mesh():
    return plsc.VectorSubcoreMesh(core_axis_name="c", subcore_axis_name="s")


@functools.partial(jax.jit, static_argnames=("parts", "c2p", "chunks"))
def _sc_scatter_rows(ytab, srcg, dstg, zeros, *, parts, c2p, chunks):
    ppc = parts // _CORES

    @functools.partial(
        pl.kernel,
        mesh=_sc_mesh(),
        compiler_params=pltpu.CompilerParams(use_tc_tiling_on_sc=False),
        out_type=jax.ShapeDtypeStruct((parts, _NP, c2p), _F32),
        scratch_types=[
            pltpu.VMEM((chunks, _K), jnp.int32),
            pltpu.VMEM((chunks, _K), jnp.int32),
            pltpu.VMEM((_K, c2p), _F32),
            pltpu.VMEM_SHARED((_NP, c2p), _F32),
            pltpu.SemaphoreType.DMA,
        ],
    )
    def k(y_hbm, src_hbm, dst_hbm, z_hbm, out_hbm, src_v, dst_v, gbuf, acc_sh, sem):
        c = lax.axis_index("c")
        s = lax.axis_index("s")
        pltpu.sync_copy(dst_hbm.at[s], dst_v)
        r0 = s * _RPT
        for p in range(ppc):
            part = c * ppc + p
            pltpu.sync_copy(src_hbm.at[part, s], src_v)
            pltpu.sync_copy(z_hbm.at[pl.ds(r0, _RPT)],
                            acc_sh.at[pl.ds(r0, _RPT)])
            plsc.subcore_barrier()

            def body(j, carry):
                pltpu.async_copy(y_hbm.at[src_v.at[j]], gbuf, sem).wait()
                pltpu.sync_copy(gbuf, acc_sh.at[dst_v.at[j]], add=True)
                return carry

            lax.fori_loop(0, chunks, body, 0)
            plsc.subcore_barrier()
            pltpu.sync_copy(acc_sh.at[pl.ds(r0, _RPT)],
                            out_hbm.at[part, pl.ds(r0, _RPT)])

    return k(ytab, srcg, dstg, zeros)


@jax.jit
def _sc_degree(dstd, ones8, zeros8):
    chunks = dstd.shape[2]

    @functools.partial(
        pl.kernel,
        mesh=_sc_mesh(),
        compiler_params=pltpu.CompilerParams(use_tc_tiling_on_sc=False),
        out_type=jax.ShapeDtypeStruct((_CORES, _NP, 8), _F32),
        scratch_types=[
            pltpu.VMEM((chunks, _K), jnp.int32),
            pltpu.VMEM((_K, 8), _F32),
            pltpu.VMEM_SHARED((_NP, 8), _F32),
        ],
    )
    def k(dst_hbm, ones_hbm, z_hbm, out_hbm, dst_v, obuf, acc_sh):
        c = lax.axis_index("c")
        s = lax.axis_index("s")
        pltpu.sync_copy(dst_hbm.at[c, s], dst_v)
        pltpu.sync_copy(ones_hbm, obuf)
        r0 = s * _RPT
        pltpu.sync_copy(z_hbm.at[pl.ds(r0, _RPT)], acc_sh.at[pl.ds(r0, _RPT)])
        plsc.subcore_barrier()

        def body(j, carry):
            pltpu.sync_copy(obuf, acc_sh.at[dst_v.at[j]], add=True)
            return carry

        lax.fori_loop(0, chunks, body, 0)
        plsc.subcore_barrier()
        pltpu.sync_copy(acc_sh.at[pl.ds(r0, _RPT)],
                        out_hbm.at[c, pl.ds(r0, _RPT)])

    return k(dstd, ones8, zeros8)


def _row_spec(c):
    return pl.BlockSpec((_BLK, c), lambda i: (i, 0))


def _half_spec(c2, parts=_CORES):
    return pl.BlockSpec((parts, _BLK, c2), lambda i: (0, i, 0))


def _full(shape):
    nd = len(shape)
    return pl.BlockSpec(shape, lambda i, _nd=nd: (0,) * _nd)


def _deg_spec():
    return pl.BlockSpec((_CORES, _BLK, 8), lambda i: (0, i, 0))


def _store_halves(ref, y):
    parts, _, c2 = ref.shape
    for q in range(parts):
        ref[q] = y[:, q * c2:(q + 1) * c2]


def _combine(s_ref, y_ref):
    parts = s_ref.shape[0]
    return jnp.concatenate([s_ref[q] + y_ref[q] for q in range(parts)], axis=1)


def _k1_body(x_ref, win_ref, bin_ref, g_ref, b_ref, w1_ref, deg_ref, y_ref):
    dinv = _dinv_of(deg_ref)
    h = jax.nn.relu(_ln(_dot(x_ref[...], win_ref[...]) + bin_ref[...],
                        g_ref[...], b_ref[...]))
    y = _dot(h, w1_ref[...]) * dinv
    _store_halves(y_ref, y)


def _mid_body(s_ref, y_ref, bias_ref, g_ref, b_ref, w_ref, deg_ref, o_ref):
    dinv = _dinv_of(deg_ref)
    t = _combine(s_ref, y_ref) * dinv + bias_ref[...]
    h = jax.nn.relu(_ln(t, g_ref[...], b_ref[...]))
    y = _dot(h, w_ref[...]) * dinv
    _store_halves(o_ref, y)


def _last_body(s_ref, y_ref, bias_ref, g_ref, b_ref, deg_ref, h_ref):
    dinv = _dinv_of(deg_ref)
    t = _combine(s_ref, y_ref) * dinv + bias_ref[...]
    h_ref[...] = jax.nn.relu(_ln(t, g_ref[...], b_ref[...]))


def _pool_body(h_ref, bat_ref, wo_ref, bo_ref, o_ref):
    iota = lax.broadcasted_iota(jnp.int32, (_NG, _N), 0).astype(_F32)
    oh = (iota == bat_ref[...]).astype(_F32)
    sums = _dot(oh, h_ref[...])
    counts = jnp.sum(oh, axis=1, keepdims=True)
    pooled = sums / jnp.maximum(counts, 1.0)
    o_ref[...] = _dot(pooled, wo_ref[...]) + bo_ref[...]


def _tc_stage1(x, W_in, b_in, g, b, W1, deg8):
    return pl.pallas_call(
        _k1_body,
        grid=(_GRID,),
        in_specs=[_row_spec(128), _full((128, 32)), _full((32,)), _full((32,)),
                  _full((32,)), _full((32, 64)), _deg_spec()],
        out_specs=_half_spec(32),
        out_shape=jax.ShapeDtypeStruct((_CORES, _N, 32), _F32),
    )(x, W_in, b_in, g, b, W1, deg8)


def _tc_mid(S, y, bias, g, b, W, deg8, parts_in, c2, parts_out, c2n):
    cin = parts_in * c2
    cout = parts_out * c2n
    return pl.pallas_call(
        _mid_body,
        grid=(_GRID,),
        in_specs=[_half_spec(c2, parts_in), _half_spec(c2, parts_in),
                  _full((cin,)), _full((cin,)), _full((cin,)),
                  _full((cin, cout)), _deg_spec()],
        out_specs=_half_spec(c2n, parts_out),
        out_shape=jax.ShapeDtypeStruct((parts_out, _N, c2n), _F32),
    )(S, y, bias, g, b, W, deg8)


def _tc_last(S, y, bias, g, b, deg8):
    return pl.pallas_call(
        _last_body,
        grid=(_GRID,),
        in_specs=[_half_spec(64, 4), _half_spec(64, 4), _full((256,)),
                  _full((256,)), _full((256,)), _deg_spec()],
        out_specs=_row_spec(256),
        out_shape=jax.ShapeDtypeStruct((_N, 256), _F32),
    )(S, y, bias, g, b, deg8)


def _tc_pool(h3, batch_f, Wo, bo):
    return pl.pallas_call(
        _pool_body,
        in_specs=[pl.BlockSpec((_N, 256), lambda: (0, 0)),
                  pl.BlockSpec((1, _N), lambda: (0, 0)),
                  pl.BlockSpec((256, 64), lambda: (0, 0)),
                  pl.BlockSpec((64,), lambda: (0,))],
        out_specs=pl.BlockSpec((_NG, 64), lambda: (0, 0)),
        out_shape=jax.ShapeDtypeStruct((_NG, 64), _F32),
    )(h3, batch_f, Wo, bo)


def kernel(x, edge_index, batch, W_in, b_in, ln_in_g, ln_in_b, W1, b1, n1_g,
           n1_b, W2, b2, n2_g, n2_b, W3, b3, n3_g, n3_b, Wo, bo):
    src = edge_index[0].astype(jnp.int32)
    dst = edge_index[1].astype(jnp.int32)

    conv_chunks = _E // (_TILES * _K)
    deg_chunks = _E // (_CORES * _TILES * _K)

    srcg2 = jnp.stack([src, src + _N]).reshape(2, _TILES, conv_chunks, _K)
    srcg4 = jnp.stack([src, src + _N, src + 2 * _N, src + 3 * _N]
                      ).reshape(4, _TILES, conv_chunks, _K)
    dstg = dst.reshape(_TILES, conv_chunks, _K)
    dstd = dst.reshape(_CORES, _TILES, deg_chunks, _K)

    z8 = jnp.zeros((_NP, 8), _F32)
    ones8 = jnp.ones((_K, 8), _F32)
    zeros = {c2: jnp.zeros((_NP, c2), _F32) for c2 in (32, 64)}
    batch_f = batch.astype(_F32).reshape(1, _N)

    deg8 = _sc_degree(dstd, ones8, z8)[:, :_N, :]

    y1 = _tc_stage1(x, W_in, b_in, ln_in_g, ln_in_b, W1, deg8)
    S1 = _sc_scatter_rows(y1.reshape(2 * _N, 32), srcg2, dstg, zeros[32],
                          parts=2, c2p=32, chunks=conv_chunks)[:, :_N, :]

    y2 = _tc_mid(S1, y1, b1, n1_g, n1_b, W2, deg8, 2, 32, 2, 64)
    S2 = _sc_scatter_rows(y2.reshape(2 * _N, 64), srcg2, dstg, zeros[64],
                          parts=2, c2p=64, chunks=conv_chunks)[:, :_N, :]

    y3 = _tc_mid(S2, y2, b2, n2_g, n2_b, W3, deg8, 2, 64, 4, 64)
    S3 = _sc_scatter_rows(y3.reshape(4 * _N, 64), srcg4, dstg, zeros[64],
                          parts=4, c2p=64, chunks=conv_chunks)[:, :_N, :]

    h3 = _tc_last(S3, y3, b3, n3_g, n3_b, deg8)
    return _tc_pool(h3, batch_f, Wo, bo)


# speedup vs baseline: 19.4154x; 1.6758x over previous
import functools

import jax
import jax.numpy as jnp
from jax import lax
from jax.experimental import pallas as pl
from jax.experimental.pallas import tpu as pltpu
from jax.experimental.pallas import tpu_sc as plsc

_N = 10000
_E = 320000
_NG = 16
_TILES = 16
_CORES = 2
_K = 125
_NP = 10240
_RPT = _NP // _TILES
_BLK = 1000
_GRID = _N // _BLK

_F32 = jnp.float32
_HI = lax.Precision.HIGHEST


def _dot(a, b):
    return lax.dot_general(a, b, (((a.ndim - 1,), (0,)), ((), ())),
                           precision=_HI, preferred_element_type=_F32)


def _ln(t, g, b, eps=1e-5):
    mu = jnp.mean(t, axis=-1, keepdims=True)
    var = jnp.mean((t - mu) ** 2, axis=-1, keepdims=True)
    return (t - mu) * lax.rsqrt(var + eps) * g + b


def _dinv_of(deg_ref):
    return lax.rsqrt(deg_ref[0, :, 0:1] + deg_ref[1, :, 0:1] + 1.0)


def _sc_mesh():
    return plsc.VectorSubcoreMesh(core_axis_name="c", subcore_axis_name="s")


@functools.partial(jax.jit, static_argnames=("parts", "c2p", "chunks"))
def _sc_scatter_rows(ytab, srcg, dstg, zeros, *, parts, c2p, chunks):
    ppc = parts // _CORES
    nb = 4
    dd = 2
    groups = chunks // nb + 1

    @functools.partial(
        pl.kernel,
        mesh=_sc_mesh(),
        compiler_params=pltpu.CompilerParams(use_tc_tiling_on_sc=False),
        out_type=jax.ShapeDtypeStruct((parts, _NP, c2p), _F32),
        scratch_types=[
            pltpu.VMEM((chunks, _K), jnp.int32),
            pltpu.VMEM((chunks, _K), jnp.int32),
        ]
        + [pltpu.VMEM((_K, c2p), _F32) for _ in range(nb)]
        + [pltpu.SemaphoreType.DMA for _ in range(2 * nb)]
        + [pltpu.VMEM_SHARED((_NP, c2p), _F32)],
    )
    def k(y_hbm, src_hbm, dst_hbm, z_hbm, out_hbm, src_v, dst_v, *rest):
        gbufs = rest[:nb]
        gsems = rest[nb:2 * nb]
        ssems = rest[2 * nb:3 * nb]
        acc_sh = rest[3 * nb]
        c = lax.axis_index("c")
        s = lax.axis_index("s")
        pltpu.sync_copy(dst_hbm.at[s], dst_v)
        r0 = s * _RPT

        def gather_start(j, b):
            pltpu.async_copy(y_hbm.at[src_v.at[j]], gbufs[b], gsems[b])

        def gather_wait(b):
            pltpu.make_async_copy(y_hbm.at[src_v.at[0]], gbufs[b],
                                  gsems[b]).wait()

        def scatter_start(j, b):
            pltpu.async_copy(gbufs[b], acc_sh.at[dst_v.at[j]], ssems[b],
                             add=True)

        def scatter_wait(b):
            pltpu.make_async_copy(gbufs[b], acc_sh.at[dst_v.at[0]],
                                  ssems[b]).wait()

        for p in range(ppc):
            part = c * ppc + p
            pltpu.sync_copy(src_hbm.at[part, s], src_v)
            pltpu.sync_copy(z_hbm.at[pl.ds(r0, _RPT)],
                            acc_sh.at[pl.ds(r0, _RPT)])
            plsc.subcore_barrier()

            def body(g, carry):
                for b in range(nb):
                    j = g * nb + b
                    pl.when(g >= 1)(lambda b=b: scatter_wait(b))
                    pl.when(g < groups - 1)(lambda j=j, b=b: gather_start(j, b))
                    b2 = (b - dd) % nb
                    jd = j - dd

                    def drain(jd=jd, b2=b2):
                        gather_wait(b2)
                        scatter_start(jd, b2)

                    if b < dd:
                        pl.when(g >= 1)(drain)
                    else:
                        pl.when(g < groups - 1)(drain)
                return carry

            lax.fori_loop(0, groups, body, 0)
            plsc.subcore_barrier()
            pltpu.sync_copy(acc_sh.at[pl.ds(r0, _RPT)],
                            out_hbm.at[part, pl.ds(r0, _RPT)])

    return k(ytab, srcg, dstg, zeros)


@jax.jit
def _sc_degree(dstd, ones8, zeros8):
    chunks = dstd.shape[2]

    @functools.partial(
        pl.kernel,
        mesh=_sc_mesh(),
        compiler_params=pltpu.CompilerParams(use_tc_tiling_on_sc=False),
        out_type=jax.ShapeDtypeStruct((_CORES, _NP, 8), _F32),
        scratch_types=[
            pltpu.VMEM((chunks, _K), jnp.int32),
            pltpu.VMEM((_K, 8), _F32),
            pltpu.VMEM_SHARED((_NP, 8), _F32),
        ],
    )
    def k(dst_hbm, ones_hbm, z_hbm, out_hbm, dst_v, obuf, acc_sh):
        c = lax.axis_index("c")
        s = lax.axis_index("s")
        pltpu.sync_copy(dst_hbm.at[c, s], dst_v)
        pltpu.sync_copy(ones_hbm, obuf)
        r0 = s * _RPT
        pltpu.sync_copy(z_hbm.at[pl.ds(r0, _RPT)], acc_sh.at[pl.ds(r0, _RPT)])
        plsc.subcore_barrier()

        def body(j, carry):
            pltpu.sync_copy(obuf, acc_sh.at[dst_v.at[j]], add=True)
            return carry

        lax.fori_loop(0, chunks, body, 0)
        plsc.subcore_barrier()
        pltpu.sync_copy(acc_sh.at[pl.ds(r0, _RPT)],
                        out_hbm.at[c, pl.ds(r0, _RPT)])

    return k(dstd, ones8, zeros8)


def _row_spec(c):
    return pl.BlockSpec((_BLK, c), lambda i: (i, 0))


def _half_spec(c2, parts=_CORES):
    return pl.BlockSpec((parts, _BLK, c2), lambda i: (0, i, 0))


def _full(shape):
    nd = len(shape)
    return pl.BlockSpec(shape, lambda i, _nd=nd: (0,) * _nd)


def _deg_spec():
    return pl.BlockSpec((_CORES, _BLK, 8), lambda i: (0, i, 0))


def _store_halves(ref, y):
    parts, _, c2 = ref.shape
    for q in range(parts):
        ref[q] = y[:, q * c2:(q + 1) * c2]


def _combine(s_ref, y_ref):
    parts = s_ref.shape[0]
    return jnp.concatenate([s_ref[q] + y_ref[q] for q in range(parts)], axis=1)


def _k1_body(x_ref, win_ref, bin_ref, g_ref, b_ref, w1_ref, deg_ref, y_ref):
    dinv = _dinv_of(deg_ref)
    h = jax.nn.relu(_ln(_dot(x_ref[...], win_ref[...]) + bin_ref[...],
                        g_ref[...], b_ref[...]))
    y = _dot(h, w1_ref[...]) * dinv
    _store_halves(y_ref, y)


def _mid_body(s_ref, y_ref, bias_ref, g_ref, b_ref, w_ref, deg_ref, o_ref):
    dinv = _dinv_of(deg_ref)
    t = _combine(s_ref, y_ref) * dinv + bias_ref[...]
    h = jax.nn.relu(_ln(t, g_ref[...], b_ref[...]))
    y = _dot(h, w_ref[...]) * dinv
    _store_halves(o_ref, y)


def _last_body(s_ref, y_ref, bias_ref, g_ref, b_ref, deg_ref, h_ref):
    dinv = _dinv_of(deg_ref)
    t = _combine(s_ref, y_ref) * dinv + bias_ref[...]
    h_ref[...] = jax.nn.relu(_ln(t, g_ref[...], b_ref[...]))


def _pool_body(h_ref, bat_ref, wo_ref, bo_ref, o_ref):
    iota = lax.broadcasted_iota(jnp.int32, (_NG, _N), 0).astype(_F32)
    oh = (iota == bat_ref[...]).astype(_F32)
    sums = _dot(oh, h_ref[...])
    counts = jnp.sum(oh, axis=1, keepdims=True)
    pooled = sums / jnp.maximum(counts, 1.0)
    o_ref[...] = _dot(pooled, wo_ref[...]) + bo_ref[...]


def _tc_stage1(x, W_in, b_in, g, b, W1, deg8):
    return pl.pallas_call(
        _k1_body,
        grid=(_GRID,),
        in_specs=[_row_spec(128), _full((128, 32)), _full((32,)), _full((32,)),
                  _full((32,)), _full((32, 64)), _deg_spec()],
        out_specs=_half_spec(32),
        out_shape=jax.ShapeDtypeStruct((_CORES, _N, 32), _F32),
    )(x, W_in, b_in, g, b, W1, deg8)


def _tc_mid(S, y, bias, g, b, W, deg8, parts_in, c2, parts_out, c2n):
    cin = parts_in * c2
    cout = parts_out * c2n
    return pl.pallas_call(
        _mid_body,
        grid=(_GRID,),
        in_specs=[_half_spec(c2, parts_in), _half_spec(c2, parts_in),
                  _full((cin,)), _full((cin,)), _full((cin,)),
                  _full((cin, cout)), _deg_spec()],
        out_specs=_half_spec(c2n, parts_out),
        out_shape=jax.ShapeDtypeStruct((parts_out, _N, c2n), _F32),
    )(S, y, bias, g, b, W, deg8)


def _tc_last(S, y, bias, g, b, deg8):
    return pl.pallas_call(
        _last_body,
        grid=(_GRID,),
        in_specs=[_half_spec(64, 4), _half_spec(64, 4), _full((256,)),
                  _full((256,)), _full((256,)), _deg_spec()],
        out_specs=_row_spec(256),
        out_shape=jax.ShapeDtypeStruct((_N, 256), _F32),
    )(S, y, bias, g, b, deg8)


def _tc_pool(h3, batch_f, Wo, bo):
    return pl.pallas_call(
        _pool_body,
        in_specs=[pl.BlockSpec((_N, 256), lambda: (0, 0)),
                  pl.BlockSpec((1, _N), lambda: (0, 0)),
                  pl.BlockSpec((256, 64), lambda: (0, 0)),
                  pl.BlockSpec((64,), lambda: (0,))],
        out_specs=pl.BlockSpec((_NG, 64), lambda: (0, 0)),
        out_shape=jax.ShapeDtypeStruct((_NG, 64), _F32),
    )(h3, batch_f, Wo, bo)


def kernel(x, edge_index, batch, W_in, b_in, ln_in_g, ln_in_b, W1, b1, n1_g,
           n1_b, W2, b2, n2_g, n2_b, W3, b3, n3_g, n3_b, Wo, bo):
    src = edge_index[0].astype(jnp.int32)
    dst = edge_index[1].astype(jnp.int32)

    conv_chunks = _E // (_TILES * _K)
    deg_chunks = _E // (_CORES * _TILES * _K)

    srcg2 = jnp.stack([src, src + _N]).reshape(2, _TILES, conv_chunks, _K)
    srcg4 = jnp.stack([src, src + _N, src + 2 * _N, src + 3 * _N]
                      ).reshape(4, _TILES, conv_chunks, _K)
    dstg = dst.reshape(_TILES, conv_chunks, _K)
    dstd = dst.reshape(_CORES, _TILES, deg_chunks, _K)

    z8 = jnp.zeros((_NP, 8), _F32)
    ones8 = jnp.ones((_K, 8), _F32)
    zeros = {c2: jnp.zeros((_NP, c2), _F32) for c2 in (32, 64)}
    batch_f = batch.astype(_F32).reshape(1, _N)

    deg8 = _sc_degree(dstd, ones8, z8)[:, :_N, :]

    y1 = _tc_stage1(x, W_in, b_in, ln_in_g, ln_in_b, W1, deg8)
    S1 = _sc_scatter_rows(y1.reshape(2 * _N, 32), srcg2, dstg, zeros[32],
                          parts=2, c2p=32, chunks=conv_chunks)[:, :_N, :]

    y2 = _tc_mid(S1, y1, b1, n1_g, n1_b, W2, deg8, 2, 32, 2, 64)
    S2 = _sc_scatter_rows(y2.reshape(2 * _N, 64), srcg2, dstg, zeros[64],
                          parts=2, c2p=64, chunks=conv_chunks)[:, :_N, :]

    y3 = _tc_mid(S2, y2, b2, n2_g, n2_b, W3, deg8, 2, 64, 4, 64)
    S3 = _sc_scatter_rows(y3.reshape(4 * _N, 64), srcg4, dstg, zeros[64],
                          parts=4, c2p=64, chunks=conv_chunks)[:, :_N, :]

    h3 = _tc_last(S3, y3, b3, n3_g, n3_b, deg8)
    return _tc_pool(h3, batch_f, Wo, bo)


# speedup vs baseline: 20.9125x; 1.0771x over previous
import functools

import jax
import jax.numpy as jnp
from jax import lax
from jax.experimental import pallas as pl
from jax.experimental.pallas import tpu as pltpu
from jax.experimental.pallas import tpu_sc as plsc

_N = 10000
_E = 320000
_NG = 16
_TILES = 16
_CORES = 2
_K = 125
_NP = 10240
_RPT = _NP // _TILES
_BLK = 1000
_GRID = _N // _BLK

_F32 = jnp.float32
_HI = lax.Precision.HIGHEST


def _dot(a, b, dims=None):
    dn = (((a.ndim - 1,), (0,)), ((), ())) if dims is None else dims
    return lax.dot_general(a, b, dn, precision=_HI,
                           preferred_element_type=_F32)


def _ln(t, g, b, eps=1e-5):
    mu = jnp.mean(t, axis=-1, keepdims=True)
    var = jnp.mean((t - mu) ** 2, axis=-1, keepdims=True)
    return (t - mu) * lax.rsqrt(var + eps) * g + b


def _sc_mesh():
    return plsc.VectorSubcoreMesh(core_axis_name="c", subcore_axis_name="s")


@functools.partial(jax.jit, static_argnames=("parts", "c2p", "chunks"))
def _sc_scatter_rows(ytabs, srcg, dstg, zeros, *, parts, c2p, chunks):
    ppc = parts // _CORES
    nb = 4
    dd = 2
    groups = chunks // nb + 1

    @functools.partial(
        pl.kernel,
        mesh=_sc_mesh(),
        compiler_params=pltpu.CompilerParams(use_tc_tiling_on_sc=False),
        out_type=jax.ShapeDtypeStruct((parts, _NP, c2p), _F32),
        scratch_types=[
            pltpu.VMEM((chunks, _K), jnp.int32),
            pltpu.VMEM((chunks, _K), jnp.int32),
        ]
        + [pltpu.VMEM((_K, c2p), _F32) for _ in range(nb)]
        + [pltpu.SemaphoreType.DMA for _ in range(2 * nb)]
        + [pltpu.VMEM_SHARED((_NP, c2p), _F32)],
    )
    def k(*refs):
        tabs = refs[:parts]
        src_hbm, dst_hbm, z_hbm, out_hbm, src_v, dst_v = refs[parts:parts + 6]
        rest = refs[parts + 6:]
        gbufs = rest[:nb]
        gsems = rest[nb:2 * nb]
        ssems = rest[2 * nb:3 * nb]
        acc_sh = rest[3 * nb]
        c = lax.axis_index("c")
        s = lax.axis_index("s")
        pltpu.sync_copy(src_hbm.at[s], src_v)
        pltpu.sync_copy(dst_hbm.at[s], dst_v)
        r0 = s * _RPT

        def gather_wait(b):
            pltpu.make_async_copy(tabs[0].at[src_v.at[0]], gbufs[b],
                                  gsems[b]).wait()

        def scatter_start(j, b):
            pltpu.async_copy(gbufs[b], acc_sh.at[dst_v.at[j]], ssems[b],
                             add=True)

        def scatter_wait(b):
            pltpu.make_async_copy(gbufs[b], acc_sh.at[dst_v.at[0]],
                                  ssems[b]).wait()

        for p in range(ppc):
            def gather_start(j, b, p=p):
                def g0():
                    pltpu.async_copy(tabs[p].at[src_v.at[j]], gbufs[b],
                                     gsems[b])

                def g1():
                    pltpu.async_copy(tabs[ppc + p].at[src_v.at[j]], gbufs[b],
                                     gsems[b])

                pl.when(c == 0)(g0)
                pl.when(c == 1)(g1)

            part = c * ppc + p
            pltpu.sync_copy(z_hbm.at[pl.ds(r0, _RPT)],
                            acc_sh.at[pl.ds(r0, _RPT)])
            plsc.subcore_barrier()

            def body(g, carry):
                for b in range(nb):
                    j = g * nb + b
                    pl.when(g >= 1)(lambda b=b: scatter_wait(b))
                    pl.when(g < groups - 1)(lambda j=j, b=b: gather_start(j, b))
                    b2 = (b - dd) % nb
                    jd = j - dd

                    def drain(jd=jd, b2=b2):
                        gather_wait(b2)
                        scatter_start(jd, b2)

                    if b < dd:
                        pl.when(g >= 1)(drain)
                    else:
                        pl.when(g < groups - 1)(drain)
                return carry

            lax.fori_loop(0, groups, body, 0)
            plsc.subcore_barrier()
            pltpu.sync_copy(acc_sh.at[pl.ds(r0, _RPT)],
                            out_hbm.at[part, pl.ds(r0, _RPT)])

    return k(*ytabs, srcg, dstg, zeros)


@jax.jit
def _sc_degree(dstd, ones8, zeros8):
    chunks = dstd.shape[2]

    @functools.partial(
        pl.kernel,
        mesh=_sc_mesh(),
        compiler_params=pltpu.CompilerParams(use_tc_tiling_on_sc=False),
        out_type=jax.ShapeDtypeStruct((_CORES, _NP, 8), _F32),
        scratch_types=[
            pltpu.VMEM((chunks, _K), jnp.int32),
            pltpu.VMEM((_K, 8), _F32),
            pltpu.VMEM_SHARED((_NP, 8), _F32),
        ],
    )
    def k(dst_hbm, ones_hbm, z_hbm, out_hbm, dst_v, obuf, acc_sh):
        c = lax.axis_index("c")
        s = lax.axis_index("s")
        pltpu.sync_copy(dst_hbm.at[c, s], dst_v)
        pltpu.sync_copy(ones_hbm, obuf)
        r0 = s * _RPT
        pltpu.sync_copy(z_hbm.at[pl.ds(r0, _RPT)], acc_sh.at[pl.ds(r0, _RPT)])
        plsc.subcore_barrier()

        def body(j, carry):
            pltpu.sync_copy(obuf, acc_sh.at[dst_v.at[j]], add=True)
            return carry

        lax.fori_loop(0, chunks, body, 0)
        plsc.subcore_barrier()
        pltpu.sync_copy(acc_sh.at[pl.ds(r0, _RPT)],
                        out_hbm.at[c, pl.ds(r0, _RPT)])

    return k(dstd, ones8, zeros8)


def _row_spec(c):
    return pl.BlockSpec((_BLK, c), lambda i: (i, 0))


def _part_spec(c2, parts):
    return pl.BlockSpec((parts, _BLK, c2), lambda i: (0, i, 0))


def _full(shape):
    nd = len(shape)
    return pl.BlockSpec(shape, lambda i, _nd=nd: (0,) * _nd)


def _dinv_body(deg_ref, o_ref):
    dv = lax.rsqrt(deg_ref[0, :, 0:1] + deg_ref[1, :, 0:1] + 1.0)
    o_ref[...] = jnp.broadcast_to(dv, (_BLK, 128))


def _tc_dinv(deg8):
    return pl.pallas_call(
        _dinv_body,
        grid=(_GRID,),
        in_specs=[pl.BlockSpec((_CORES, _BLK, 8), lambda i: (0, i, 0))],
        out_specs=_row_spec(128),
        out_shape=jax.ShapeDtypeStruct((_N, 128), _F32),
    )(deg8)


def _k1_body(x_ref, win_ref, bin_ref, g_ref, b_ref, w1_ref, dv_ref, *y_refs):
    dinv = dv_ref[:, 0:1]
    h = jax.nn.relu(_ln(_dot(x_ref[...], win_ref[...]) + bin_ref[...],
                        g_ref[...], b_ref[...]))
    y = _dot(h, w1_ref[...]) * dinv
    c2 = y_refs[0].shape[1]
    for q, r in enumerate(y_refs):
        r[...] = y[:, q * c2:(q + 1) * c2]


def _mid_body(s_ref, bias_ref, g_ref, b_ref, w_ref, dv_ref, *y_io):
    parts_in = s_ref.shape[0]
    yin = y_io[:parts_in]
    yout = y_io[parts_in:]
    dinv = dv_ref[:, 0:1]
    t = jnp.concatenate([s_ref[q] + yin[q][...] for q in range(parts_in)],
                        axis=1)
    t = t * dinv + bias_ref[...]
    h = jax.nn.relu(_ln(t, g_ref[...], b_ref[...]))
    y = _dot(h, w_ref[...]) * dinv
    c2 = yout[0].shape[1]
    for q, r in enumerate(yout):
        r[...] = y[:, q * c2:(q + 1) * c2]


def _fin_body(s_ref, bias_ref, g_ref, b_ref, dv_ref, bat_ref, wo_ref, bo_ref,
              y0, y1, y2, y3, o_ref, sums_ref, cnt_ref):
    i = pl.program_id(0)

    @pl.when(i == 0)
    def _():
        sums_ref[...] = jnp.zeros_like(sums_ref)
        cnt_ref[...] = jnp.zeros_like(cnt_ref)

    dinv = dv_ref[:, 0:1]
    yin = (y0, y1, y2, y3)
    t = jnp.concatenate([s_ref[q] + yin[q][...] for q in range(4)], axis=1)
    t = t * dinv + bias_ref[...]
    h = jax.nn.relu(_ln(t, g_ref[...], b_ref[...]))
    iota = lax.broadcasted_iota(jnp.int32, (_BLK, _NG), 1).astype(_F32)
    oht = (bat_ref[...] == iota).astype(_F32)
    sums_ref[...] += _dot(oht, h, dims=(((0,), (0,)), ((), ())))
    cnt_ref[...] += jnp.sum(oht, axis=0, keepdims=True)

    @pl.when(i == _GRID - 1)
    def _():
        recip = 1.0 / jnp.maximum(cnt_ref[...], 1.0)
        eye = (lax.broadcasted_iota(jnp.int32, (_NG, _NG), 0)
               == lax.broadcasted_iota(jnp.int32, (_NG, _NG), 1)).astype(_F32)
        pooled = _dot(eye * recip, sums_ref[...])
        o_ref[...] = _dot(pooled, wo_ref[...]) + bo_ref[...]


def _tc_stage1(x, W_in, b_in, g, b, W1, dinv):
    return pl.pallas_call(
        _k1_body,
        grid=(_GRID,),
        in_specs=[_row_spec(128), _full((128, 32)), _full((32,)), _full((32,)),
                  _full((32,)), _full((32, 64)), _row_spec(128)],
        out_specs=[_row_spec(32)] * 2,
        out_shape=[jax.ShapeDtypeStruct((_N, 32), _F32)] * 2,
    )(x, W_in, b_in, g, b, W1, dinv)


def _tc_mid(S, yin, bias, g, b, W, dinv, parts_in, c2, parts_out, c2n):
    cin = parts_in * c2
    cout = parts_out * c2n
    return pl.pallas_call(
        _mid_body,
        grid=(_GRID,),
        in_specs=[_part_spec(c2, parts_in), _full((cin,)), _full((cin,)),
                  _full((cin,)), _full((cin, cout)), _row_spec(128)]
                 + [_row_spec(c2)] * parts_in,
        out_specs=[_row_spec(c2n)] * parts_out,
        out_shape=[jax.ShapeDtypeStruct((_N, c2n), _F32)] * parts_out,
    )(S, bias, g, b, W, dinv, *yin)


def _tc_final(S, yin, bias, g, b, dinv, batch_f, Wo, bo):
    return pl.pallas_call(
        _fin_body,
        grid=(_GRID,),
        in_specs=[_part_spec(64, 4), _full((256,)), _full((256,)),
                  _full((256,)), _row_spec(128),
                  pl.BlockSpec((_BLK, 1), lambda i: (i, 0)),
                  _full((256, 64)), _full((64,))]
                 + [_row_spec(64)] * 4,
        out_specs=pl.BlockSpec((_NG, 64), lambda i: (0, 0)),
        out_shape=jax.ShapeDtypeStruct((_NG, 64), _F32),
        scratch_shapes=[pltpu.VMEM((_NG, 256), _F32),
                        pltpu.VMEM((1, _NG), _F32)],
    )(S, bias, g, b, dinv, batch_f, Wo, bo, *yin)


def kernel(x, edge_index, batch, W_in, b_in, ln_in_g, ln_in_b, W1, b1, n1_g,
           n1_b, W2, b2, n2_g, n2_b, W3, b3, n3_g, n3_b, Wo, bo):
    src = edge_index[0].astype(jnp.int32)
    dst = edge_index[1].astype(jnp.int32)

    conv_chunks = _E // (_TILES * _K)
    deg_chunks = _E // (_CORES * _TILES * _K)

    srcg = src.reshape(_TILES, conv_chunks, _K)
    dstg = dst.reshape(_TILES, conv_chunks, _K)
    dstd = dst.reshape(_CORES, _TILES, deg_chunks, _K)

    z8 = jnp.zeros((_NP, 8), _F32)
    ones8 = jnp.ones((_K, 8), _F32)
    zeros = {c2: jnp.zeros((_NP, c2), _F32) for c2 in (32, 64)}
    batch_f = batch.astype(_F32).reshape(_N, 1)

    deg8 = _sc_degree(dstd, ones8, z8)
    dinv = _tc_dinv(deg8)

    y1 = _tc_stage1(x, W_in, b_in, ln_in_g, ln_in_b, W1, dinv)
    S1 = _sc_scatter_rows(tuple(y1), srcg, dstg, zeros[32],
                          parts=2, c2p=32, chunks=conv_chunks)

    y2 = _tc_mid(S1, y1, b1, n1_g, n1_b, W2, dinv, 2, 32, 2, 64)
    S2 = _sc_scatter_rows(tuple(y2), srcg, dstg, zeros[64],
                          parts=2, c2p=64, chunks=conv_chunks)

    y3 = _tc_mid(S2, y2, b2, n2_g, n2_b, W3, dinv, 2, 64, 4, 64)
    S3 = _sc_scatter_rows(tuple(y3), srcg, dstg, zeros[64],
                          parts=4, c2p=64, chunks=conv_chunks)

    return _tc_final(S3, y3, b3, n3_g, n3_b, dinv, batch_f, Wo, bo)


# speedup vs baseline: 23.0858x; 1.1039x over previous
import functools

import jax
import jax.numpy as jnp
from jax import lax
from jax.experimental import pallas as pl
from jax.experimental.pallas import tpu as pltpu
from jax.experimental.pallas import tpu_sc as plsc

_N = 10000
_E = 320000
_NG = 16
_TILES = 16
_CORES = 2
_K = 125
_NP = 10240
_RPT = _NP // _TILES
_BLK = 1024
_GRID = _NP // _BLK

_F32 = jnp.float32
_HI = lax.Precision.HIGHEST


def _dot(a, b, dims=None):
    dn = (((a.ndim - 1,), (0,)), ((), ())) if dims is None else dims
    return lax.dot_general(a, b, dn, precision=_HI,
                           preferred_element_type=_F32)


def _ln(t, g, b, eps=1e-5):
    mu = jnp.mean(t, axis=-1, keepdims=True)
    var = jnp.mean((t - mu) ** 2, axis=-1, keepdims=True)
    return (t - mu) * lax.rsqrt(var + eps) * g + b


def _sc_mesh():
    return plsc.VectorSubcoreMesh(core_axis_name="c", subcore_axis_name="s")


@functools.partial(jax.jit, static_argnames=("parts", "c2p", "chunks"))
def _sc_scatter_rows(ytabs, srcg, dstg, zeros, *, parts, c2p, chunks):
    ppc = parts // _CORES
    gpt = 128 // c2p
    ntab = max(1, parts * c2p // 128)
    nb = 4
    dd = 2
    groups = chunks // nb + 1

    @functools.partial(
        pl.kernel,
        mesh=_sc_mesh(),
        compiler_params=pltpu.CompilerParams(use_tc_tiling_on_sc=False),
        out_type=[jax.ShapeDtypeStruct((_NP, 128), _F32)] * ntab,
        scratch_types=[
            pltpu.VMEM((chunks, _K), jnp.int32),
            pltpu.VMEM((chunks, _K), jnp.int32),
        ]
        + [pltpu.VMEM((_K, c2p), _F32) for _ in range(nb)]
        + [pltpu.SemaphoreType.DMA for _ in range(2 * nb)]
        + [pltpu.VMEM_SHARED((_NP, c2p), _F32)],
    )
    def k(*refs):
        tabs = refs[:parts]
        src_hbm, dst_hbm, z_hbm = refs[parts:parts + 3]
        outs = refs[parts + 3:parts + 3 + ntab]
        rest = refs[parts + 3 + ntab:]
        src_v, dst_v = rest[0], rest[1]
        gbufs = rest[2:2 + nb]
        gsems = rest[2 + nb:2 + 2 * nb]
        ssems = rest[2 + 2 * nb:2 + 3 * nb]
        acc_sh = rest[2 + 3 * nb]
        c = lax.axis_index("c")
        s = lax.axis_index("s")
        pltpu.sync_copy(src_hbm.at[s], src_v)
        pltpu.sync_copy(dst_hbm.at[s], dst_v)
        r0 = s * _RPT

        def gather_wait(b):
            pltpu.make_async_copy(tabs[0].at[src_v.at[0]], gbufs[b],
                                  gsems[b]).wait()

        def scatter_start(j, b):
            pltpu.async_copy(gbufs[b], acc_sh.at[dst_v.at[j]], ssems[b],
                             add=True)

        def scatter_wait(b):
            pltpu.make_async_copy(gbufs[b], acc_sh.at[dst_v.at[0]],
                                  ssems[b]).wait()

        for p in range(ppc):
            def gather_start(j, b, p=p):
                def g0():
                    pltpu.async_copy(tabs[p].at[src_v.at[j]], gbufs[b],
                                     gsems[b])

                def g1():
                    pltpu.async_copy(tabs[ppc + p].at[src_v.at[j]], gbufs[b],
                                     gsems[b])

                pl.when(c == 0)(g0)
                pl.when(c == 1)(g1)

            pltpu.sync_copy(z_hbm.at[pl.ds(r0, _RPT)],
                            acc_sh.at[pl.ds(r0, _RPT)])
            plsc.subcore_barrier()

            def body(g, carry):
                for b in range(nb):
                    j = g * nb + b
                    pl.when(g >= 1)(lambda b=b: scatter_wait(b))
                    pl.when(g < groups - 1)(lambda j=j, b=b: gather_start(j, b))
                    b2 = (b - dd) % nb
                    jd = j - dd

                    def drain(jd=jd, b2=b2):
                        gather_wait(b2)
                        scatter_start(jd, b2)

                    if b < dd:
                        pl.when(g >= 1)(drain)
                    else:
                        pl.when(g < groups - 1)(drain)
                return carry

            lax.fori_loop(0, groups, body, 0)
            plsc.subcore_barrier()

            def write_for(part):
                out = outs[part // gpt]
                coff = (part % gpt) * c2p

                def go():
                    pltpu.sync_copy(
                        acc_sh.at[pl.ds(r0, _RPT)],
                        out.at[pl.ds(r0, _RPT), pl.ds(coff, c2p)])
                return go

            pl.when(c == 0)(write_for(0 * ppc + p))
            pl.when(c == 1)(write_for(1 * ppc + p))
            plsc.subcore_barrier()

    return k(*ytabs, srcg, dstg, zeros)


@jax.jit
def _sc_degree(dstd, ones8, zeros8):
    chunks = dstd.shape[2]

    @functools.partial(
        pl.kernel,
        mesh=_sc_mesh(),
        compiler_params=pltpu.CompilerParams(use_tc_tiling_on_sc=False),
        out_type=jax.ShapeDtypeStruct((_CORES, _NP, 8), _F32),
        scratch_types=[
            pltpu.VMEM((chunks, _K), jnp.int32),
            pltpu.VMEM((_K, 8), _F32),
            pltpu.VMEM_SHARED((_NP, 8), _F32),
        ],
    )
    def k(dst_hbm, ones_hbm, z_hbm, out_hbm, dst_v, obuf, acc_sh):
        c = lax.axis_index("c")
        s = lax.axis_index("s")
        pltpu.sync_copy(dst_hbm.at[c, s], dst_v)
        pltpu.sync_copy(ones_hbm, obuf)
        r0 = s * _RPT
        pltpu.sync_copy(z_hbm.at[pl.ds(r0, _RPT)], acc_sh.at[pl.ds(r0, _RPT)])
        plsc.subcore_barrier()

        def body(j, carry):
            pltpu.sync_copy(obuf, acc_sh.at[dst_v.at[j]], add=True)
            return carry

        lax.fori_loop(0, chunks, body, 0)
        plsc.subcore_barrier()
        pltpu.sync_copy(acc_sh.at[pl.ds(r0, _RPT)],
                        out_hbm.at[c, pl.ds(r0, _RPT)])

    return k(dstd, ones8, zeros8)


def _row_spec():
    return pl.BlockSpec((_BLK, 128), lambda i: (i, 0))


def _full(shape):
    nd = len(shape)
    return pl.BlockSpec(shape, lambda i, _nd=nd: (0,) * _nd)


def _dinv_body(deg_ref, o_ref):
    dv = lax.rsqrt(deg_ref[0, :, 0:1] + deg_ref[1, :, 0:1] + 1.0)
    o_ref[...] = jnp.broadcast_to(dv, (_BLK, 128))


def _tc_dinv(deg8):
    return pl.pallas_call(
        _dinv_body,
        grid=(_GRID,),
        in_specs=[pl.BlockSpec((_CORES, _BLK, 8), lambda i: (0, i, 0))],
        out_specs=_row_spec(),
        out_shape=jax.ShapeDtypeStruct((_NP, 128), _F32),
    )(deg8)


def _k1_body(x_ref, win_ref, bin_ref, g_ref, b_ref, w1_ref, dv_ref,
             y0_ref, y1_ref):
    dinv = dv_ref[:, 0:1]
    h = jax.nn.relu(_ln(_dot(x_ref[...], win_ref[...]) + bin_ref[...],
                        g_ref[...], b_ref[...]))
    y = _dot(h, w1_ref[...]) * dinv
    y0_ref[...] = y[:, :32]
    y1_ref[...] = y[:, 32:]


def _mid2_body(s_ref, bias_ref, g_ref, b_ref, w_ref, dv_ref, *y_io, cin=64,
               parts_out=2):
    yin = y_io[:len(y_io) - parts_out]
    yout = y_io[len(y_io) - parts_out:]
    dinv = dv_ref[:, 0:1]
    yprev = jnp.concatenate([r[...] for r in yin], axis=1)
    t = (s_ref[:, :cin] + yprev) * dinv + bias_ref[...]
    h = jax.nn.relu(_ln(t, g_ref[...], b_ref[...]))
    y = _dot(h, w_ref[...]) * dinv
    c2 = y.shape[1] // parts_out
    for q, r in enumerate(yout):
        r[...] = y[:, q * c2:(q + 1) * c2]


def _fin_body(sa_ref, sb_ref, bias_ref, g_ref, b_ref, dv_ref,
              bat_ref, wo_ref, bo_ref, y0, y1, y2, y3, o_ref,
              sums_ref, cnt_ref):
    i = pl.program_id(0)

    @pl.when(i == 0)
    def _():
        sums_ref[...] = jnp.zeros_like(sums_ref)
        cnt_ref[...] = jnp.zeros_like(cnt_ref)

    dinv = dv_ref[:, 0:1]
    t = jnp.concatenate([sa_ref[...], sb_ref[...]], axis=1)
    t = t + jnp.concatenate([r[...] for r in (y0, y1, y2, y3)], axis=1)
    t = t * dinv + bias_ref[...]
    h = jax.nn.relu(_ln(t, g_ref[...], b_ref[...]))
    iota = lax.broadcasted_iota(jnp.int32, (_BLK, _NG), 1).astype(_F32)
    oht = (bat_ref[:, 0:1] == iota).astype(_F32)
    sums_ref[...] += _dot(oht, h, dims=(((0,), (0,)), ((), ())))
    cnt_ref[...] += jnp.sum(oht, axis=0, keepdims=True)

    @pl.when(i == _GRID - 1)
    def _():
        recip = 1.0 / jnp.maximum(cnt_ref[...], 1.0)
        eye = (lax.broadcasted_iota(jnp.int32, (_NG, _NG), 0)
               == lax.broadcasted_iota(jnp.int32, (_NG, _NG), 1)).astype(_F32)
        pooled = _dot(eye * recip, sums_ref[...])
        o_ref[...] = _dot(pooled, wo_ref[...]) + bo_ref[...]


def _nspec(c2):
    return pl.BlockSpec((_BLK, c2), lambda i: (i, 0))


def _tc_stage1(x, W_in, b_in, g, b, W1, dinv):
    return pl.pallas_call(
        _k1_body,
        grid=(_GRID,),
        in_specs=[_row_spec(), _full((128, 32)), _full((32,)), _full((32,)),
                  _full((32,)), _full((32, 64)), _row_spec()],
        out_specs=[_nspec(32)] * 2,
        out_shape=[jax.ShapeDtypeStruct((_NP, 32), _F32)] * 2,
    )(x, W_in, b_in, g, b, W1, dinv)


def _tc_mid(S, yin, bias, g, b, W, dinv, cin, cout, parts_out, c2n):
    body = functools.partial(_mid2_body, cin=cin, parts_out=parts_out)
    c2 = cin // len(yin)
    return pl.pallas_call(
        body,
        grid=(_GRID,),
        in_specs=[_row_spec(), _full((cin,)), _full((cin,)),
                  _full((cin,)), _full((cin, cout)), _row_spec()]
                 + [_nspec(c2)] * len(yin),
        out_specs=[_nspec(c2n)] * parts_out,
        out_shape=[jax.ShapeDtypeStruct((_NP, c2n), _F32)] * parts_out,
    )(S, bias, g, b, W, dinv, *yin)


def _tc_final(Sa, Sb, yin, bias, g, b, dinv, batch_f, Wo, bo):
    return pl.pallas_call(
        _fin_body,
        grid=(_GRID,),
        in_specs=[_row_spec(), _row_spec(),
                  _full((256,)), _full((256,)), _full((256,)), _row_spec(),
                  _row_spec(), _full((256, 64)), _full((64,))]
                 + [_nspec(64)] * 4,
        out_specs=pl.BlockSpec((_NG, 64), lambda i: (0, 0)),
        out_shape=jax.ShapeDtypeStruct((_NG, 64), _F32),
        scratch_shapes=[pltpu.VMEM((_NG, 256), _F32),
                        pltpu.VMEM((1, _NG), _F32)],
    )(Sa, Sb, bias, g, b, dinv, batch_f, Wo, bo, *yin)


def kernel(x, edge_index, batch, W_in, b_in, ln_in_g, ln_in_b, W1, b1, n1_g,
           n1_b, W2, b2, n2_g, n2_b, W3, b3, n3_g, n3_b, Wo, bo):
    src = edge_index[0].astype(jnp.int32)
    dst = edge_index[1].astype(jnp.int32)

    conv_chunks = _E // (_TILES * _K)
    deg_chunks = _E // (_CORES * _TILES * _K)

    srcg = src.reshape(_TILES, conv_chunks, _K)
    dstg = dst.reshape(_TILES, conv_chunks, _K)
    dstd = dst.reshape(_CORES, _TILES, deg_chunks, _K)

    z8 = jnp.zeros((_NP, 8), _F32)
    ones8 = jnp.ones((_K, 8), _F32)
    zeros = {c2: jnp.zeros((_NP, c2), _F32) for c2 in (32, 64)}
    x_p = jnp.pad(x, ((0, _NP - _N), (0, 0)))
    batch_f = jnp.pad(batch.astype(_F32), (0, _NP - _N),
                      constant_values=-1.0).reshape(_NP, 1)
    batch_w = jnp.broadcast_to(batch_f, (_NP, 128))

    deg8 = _sc_degree(dstd, ones8, z8)
    dinv = _tc_dinv(deg8)

    y1 = _tc_stage1(x_p, W_in, b_in, ln_in_g, ln_in_b, W1, dinv)
    (S1,) = _sc_scatter_rows(tuple(y1), srcg, dstg, zeros[32],
                             parts=2, c2p=32, chunks=conv_chunks)

    y2 = _tc_mid(S1, y1, b1, n1_g, n1_b, W2, dinv, 64, 128, 2, 64)
    (S2,) = _sc_scatter_rows(tuple(y2), srcg, dstg, zeros[64],
                             parts=2, c2p=64, chunks=conv_chunks)

    y3 = _tc_mid(S2, y2, b2, n2_g, n2_b, W3, dinv, 128, 256, 4, 64)
    S3a, S3b = _sc_scatter_rows(tuple(y3), srcg, dstg, zeros[64],
                                parts=4, c2p=64, chunks=conv_chunks)

    return _tc_final(S3a, S3b, y3, b3, n3_g, n3_b, dinv, batch_w, Wo, bo)


# speedup vs baseline: 27.2411x; 1.1800x over previous
import functools

import jax
import jax.numpy as jnp
from jax import lax
from jax.experimental import pallas as pl
from jax.experimental.pallas import tpu as pltpu
from jax.experimental.pallas import tpu_sc as plsc

_N = 10000
_E = 320000
_NG = 16
_TILES = 16
_CORES = 2
_K = 125
_NP = 10240
_RPT = _NP // _TILES
_BLK = 1024
_GRID = _NP // _BLK

_F32 = jnp.float32
_HI = lax.Precision.HIGHEST


def _dot(a, b, dims=None):
    dn = (((a.ndim - 1,), (0,)), ((), ())) if dims is None else dims
    return lax.dot_general(a, b, dn, precision=_HI,
                           preferred_element_type=_F32)


def _ln(t, g, b, eps=1e-5):
    mu = jnp.mean(t, axis=-1, keepdims=True)
    var = jnp.mean((t - mu) ** 2, axis=-1, keepdims=True)
    return (t - mu) * lax.rsqrt(var + eps) * g + b


def _sc_mesh():
    return plsc.VectorSubcoreMesh(core_axis_name="c", subcore_axis_name="s")


@functools.partial(jax.jit, static_argnames=("parts", "c2p", "chunks"))
def _sc_scatter_rows(ytabs, srcp, dstg, zeros, *, parts, c2p, chunks):
    ppc = parts // _CORES
    gpt = 128 // c2p
    ntab = max(1, parts // gpt)
    nb = 4
    dd = 2
    groups = chunks // nb + 1

    @functools.partial(
        pl.kernel,
        mesh=_sc_mesh(),
        compiler_params=pltpu.CompilerParams(use_tc_tiling_on_sc=False),
        out_type=[jax.ShapeDtypeStruct((_NP, 128), _F32)] * ntab,
        scratch_types=[
            pltpu.VMEM((chunks, _K), jnp.int32),
            pltpu.VMEM((chunks, _K), jnp.int32),
        ]
        + [pltpu.VMEM((_K, c2p), _F32) for _ in range(nb)]
        + [pltpu.SemaphoreType.DMA for _ in range(2 * nb)]
        + [pltpu.VMEM_SHARED((_NP, c2p), _F32)],
    )
    def k(*refs):
        tabs = refs[:ntab]
        srcp0_hbm, srcp1_hbm, dst_hbm, z_hbm = refs[ntab:ntab + 4]
        outs = refs[ntab + 4:2 * ntab + 4]
        rest = refs[2 * ntab + 4:]
        src_v, dst_v = rest[0], rest[1]
        gbufs = rest[2:2 + nb]
        gsems = rest[2 + nb:2 + 2 * nb]
        ssems = rest[2 + 2 * nb:2 + 3 * nb]
        acc_sh = rest[2 + 3 * nb]
        c = lax.axis_index("c")
        s = lax.axis_index("s")
        pltpu.sync_copy(dst_hbm.at[s], dst_v)
        r0 = s * _RPT

        def gather_wait(b):
            pltpu.make_async_copy(tabs[0].at[src_v.at[0]], gbufs[b],
                                  gsems[b]).wait()

        def scatter_start(j, b):
            pltpu.async_copy(gbufs[b], acc_sh.at[dst_v.at[j]], ssems[b],
                             add=True)

        def scatter_wait(b):
            pltpu.make_async_copy(gbufs[b], acc_sh.at[dst_v.at[0]],
                                  ssems[b]).wait()

        srcps = (srcp0_hbm, srcp1_hbm)
        for p in range(ppc):
            par = [(cc * ppc + p) % gpt for cc in range(_CORES)]
            tix = [(cc * ppc + p) // gpt for cc in range(_CORES)]

            if par[0] == par[1]:
                pltpu.sync_copy(srcps[par[0]].at[s], src_v)
            else:
                pl.when(c == 0)(
                    lambda: pltpu.sync_copy(srcps[par[0]].at[s], src_v))
                pl.when(c == 1)(
                    lambda: pltpu.sync_copy(srcps[par[1]].at[s], src_v))

            def gather_start(j, b, tix=tix):
                if tix[0] == tix[1]:
                    pltpu.async_copy(tabs[tix[0]].at[src_v.at[j]], gbufs[b],
                                     gsems[b])
                else:
                    def g0():
                        pltpu.async_copy(tabs[tix[0]].at[src_v.at[j]],
                                         gbufs[b], gsems[b])

                    def g1():
                        pltpu.async_copy(tabs[tix[1]].at[src_v.at[j]],
                                         gbufs[b], gsems[b])

                    pl.when(c == 0)(g0)
                    pl.when(c == 1)(g1)

            pltpu.sync_copy(z_hbm.at[pl.ds(r0, _RPT)],
                            acc_sh.at[pl.ds(r0, _RPT)])
            plsc.subcore_barrier()

            def body(g, carry):
                for b in range(nb):
                    j = g * nb + b
                    pl.when(g >= 1)(lambda b=b: scatter_wait(b))
                    pl.when(g < groups - 1)(lambda j=j, b=b: gather_start(j, b))
                    b2 = (b - dd) % nb
                    jd = j - dd

                    def drain(jd=jd, b2=b2):
                        gather_wait(b2)
                        scatter_start(jd, b2)

                    if b < dd:
                        pl.when(g >= 1)(drain)
                    else:
                        pl.when(g < groups - 1)(drain)
                return carry

            lax.fori_loop(0, groups, body, 0)
            plsc.subcore_barrier()

            def write_for(part):
                out = outs[part // gpt]
                coff = (part % gpt) * c2p

                def go():
                    pltpu.sync_copy(
                        acc_sh.at[pl.ds(r0, _RPT)],
                        out.at[pl.ds(r0, _RPT), pl.ds(coff, c2p)])
                return go

            pl.when(c == 0)(write_for(0 * ppc + p))
            pl.when(c == 1)(write_for(1 * ppc + p))
            plsc.subcore_barrier()

    return k(*ytabs, srcp[0], srcp[1], dstg, zeros)


@jax.jit
def _sc_degree(dstd, ones8, zeros8):
    chunks = dstd.shape[2]

    @functools.partial(
        pl.kernel,
        mesh=_sc_mesh(),
        compiler_params=pltpu.CompilerParams(use_tc_tiling_on_sc=False),
        out_type=jax.ShapeDtypeStruct((_NP, 128), _F32),
        scratch_types=[
            pltpu.VMEM((chunks, _K), jnp.int32),
            pltpu.VMEM((_K, 8), _F32),
            pltpu.VMEM_SHARED((_NP, 8), _F32),
        ],
    )
    def k(dst_hbm, ones_hbm, z_hbm, out_hbm, dst_v, obuf, acc_sh):
        c = lax.axis_index("c")
        s = lax.axis_index("s")
        pltpu.sync_copy(dst_hbm.at[c, s], dst_v)
        pltpu.sync_copy(ones_hbm, obuf)
        r0 = s * _RPT
        pltpu.sync_copy(z_hbm.at[pl.ds(r0, _RPT)], acc_sh.at[pl.ds(r0, _RPT)])
        plsc.subcore_barrier()

        def body(j, carry):
            pltpu.sync_copy(obuf, acc_sh.at[dst_v.at[j]], add=True)
            return carry

        lax.fori_loop(0, chunks, body, 0)
        plsc.subcore_barrier()

        def w0():
            pltpu.sync_copy(acc_sh.at[pl.ds(r0, _RPT)],
                            out_hbm.at[pl.ds(r0, _RPT), pl.ds(0, 8)])

        def w1():
            pltpu.sync_copy(acc_sh.at[pl.ds(r0, _RPT)],
                            out_hbm.at[pl.ds(r0, _RPT), pl.ds(8, 8)])

        pl.when(c == 0)(w0)
        pl.when(c == 1)(w1)

    return k(dstd, ones8, zeros8)


def _row_spec():
    return pl.BlockSpec((_BLK, 128), lambda i: (i, 0))


def _full(shape):
    nd = len(shape)
    return pl.BlockSpec(shape, lambda i, _nd=nd: (0,) * _nd)


def _k1_body(x_ref, win_ref, bin_ref, g_ref, b_ref, w1_ref, deg_ref,
             y_ref, dv_ref):
    dinv = lax.rsqrt(deg_ref[:, 0:1] + deg_ref[:, 8:9] + 1.0)
    dv_ref[...] = jnp.broadcast_to(dinv, (_BLK, 128))
    h = jax.nn.relu(_ln(_dot(x_ref[...], win_ref[...]) + bin_ref[...],
                        g_ref[...], b_ref[...]))
    y = _dot(h, w1_ref[...]) * dinv
    y_ref[...] = jnp.concatenate([y, jnp.zeros_like(y)], axis=1)


def _mid2_body(s_ref, y_ref, bias_ref, g_ref, b_ref, w_ref, dv_ref, *yout,
               cin=64):
    dinv = dv_ref[:, 0:1]
    t = (s_ref[:, :cin] + y_ref[:, :cin]) * dinv + bias_ref[...]
    h = jax.nn.relu(_ln(t, g_ref[...], b_ref[...]))
    y = _dot(h, w_ref[...]) * dinv
    for q, r in enumerate(yout):
        r[...] = y[:, q * 128:(q + 1) * 128]


def _fin_body(sa_ref, sb_ref, ya_ref, yb_ref, bias_ref, g_ref, b_ref, dv_ref,
              bat_ref, wo_ref, bo_ref, o_ref, sums_ref, cnt_ref):
    i = pl.program_id(0)

    @pl.when(i == 0)
    def _():
        sums_ref[...] = jnp.zeros_like(sums_ref)
        cnt_ref[...] = jnp.zeros_like(cnt_ref)

    dinv = dv_ref[:, 0:1]
    t = jnp.concatenate([sa_ref[...] + ya_ref[...],
                         sb_ref[...] + yb_ref[...]], axis=1)
    t = t * dinv + bias_ref[...]
    h = jax.nn.relu(_ln(t, g_ref[...], b_ref[...]))
    iota = lax.broadcasted_iota(jnp.int32, (_BLK, _NG), 1).astype(_F32)
    oht = (bat_ref[:, 0:1] == iota).astype(_F32)
    sums_ref[...] += _dot(oht, h, dims=(((0,), (0,)), ((), ())))
    cnt_ref[...] += jnp.sum(oht, axis=0, keepdims=True)

    @pl.when(i == _GRID - 1)
    def _():
        recip = 1.0 / jnp.maximum(cnt_ref[...], 1.0)
        eye = (lax.broadcasted_iota(jnp.int32, (_NG, _NG), 0)
               == lax.broadcasted_iota(jnp.int32, (_NG, _NG), 1)).astype(_F32)
        pooled = _dot(eye * recip, sums_ref[...])
        o_ref[...] = _dot(pooled, wo_ref[...]) + bo_ref[...]


def _tc_stage1(x, W_in, b_in, g, b, W1, deg):
    return pl.pallas_call(
        _k1_body,
        grid=(_GRID,),
        in_specs=[_row_spec(), _full((128, 32)), _full((32,)), _full((32,)),
                  _full((32,)), _full((32, 64)), _row_spec()],
        out_specs=[_row_spec()] * 2,
        out_shape=[jax.ShapeDtypeStruct((_NP, 128), _F32)] * 2,
    )(x, W_in, b_in, g, b, W1, deg)


def _tc_mid(S, y, bias, g, b, W, dinv, cin, cout):
    body = functools.partial(_mid2_body, cin=cin)
    nout = cout // 128
    return pl.pallas_call(
        body,
        grid=(_GRID,),
        in_specs=[_row_spec(), _row_spec(), _full((cin,)), _full((cin,)),
                  _full((cin,)), _full((cin, cout)), _row_spec()],
        out_specs=[_row_spec()] * nout,
        out_shape=[jax.ShapeDtypeStruct((_NP, 128), _F32)] * nout,
    )(S, y, bias, g, b, W, dinv)


def _tc_final(Sa, Sb, ya, yb, bias, g, b, dinv, batch_f, Wo, bo):
    return pl.pallas_call(
        _fin_body,
        grid=(_GRID,),
        in_specs=[_row_spec(), _row_spec(), _row_spec(), _row_spec(),
                  _full((256,)), _full((256,)), _full((256,)), _row_spec(),
                  _row_spec(), _full((256, 64)), _full((64,))],
        out_specs=pl.BlockSpec((_NG, 64), lambda i: (0, 0)),
        out_shape=jax.ShapeDtypeStruct((_NG, 64), _F32),
        scratch_shapes=[pltpu.VMEM((_NG, 256), _F32),
                        pltpu.VMEM((1, _NG), _F32)],
    )(Sa, Sb, ya, yb, bias, g, b, dinv, batch_f, Wo, bo)


def kernel(x, edge_index, batch, W_in, b_in, ln_in_g, ln_in_b, W1, b1, n1_g,
           n1_b, W2, b2, n2_g, n2_b, W3, b3, n3_g, n3_b, Wo, bo):
    src = edge_index[0].astype(jnp.int32)
    dst = edge_index[1].astype(jnp.int32)

    conv_chunks = _E // (_TILES * _K)
    deg_chunks = _E // (_CORES * _TILES * _K)

    def idx(mult, q):
        return (mult * src + q).reshape(_TILES, conv_chunks, _K)

    src2 = (idx(2, 0), idx(2, 1))
    src4 = (idx(4, 0), idx(4, 1))
    dstg = dst.reshape(_TILES, conv_chunks, _K)
    dstd = dst.reshape(_CORES, _TILES, deg_chunks, _K)

    z8 = jnp.zeros((_NP, 8), _F32)
    ones8 = jnp.ones((_K, 8), _F32)
    zeros = {c2: jnp.zeros((_NP, c2), _F32) for c2 in (32, 64)}
    x_p = jnp.pad(x, ((0, _NP - _N), (0, 0)))
    batch_f = jnp.pad(batch.astype(_F32), (0, _NP - _N),
                      constant_values=-1.0).reshape(_NP, 1)
    batch_w = jnp.broadcast_to(batch_f, (_NP, 128))

    deg = _sc_degree(dstd, ones8, z8)

    y1, dinv = _tc_stage1(x_p, W_in, b_in, ln_in_g, ln_in_b, W1, deg)
    (S1,) = _sc_scatter_rows((y1.reshape(4 * _NP, 32),), src4, dstg,
                             zeros[32], parts=2, c2p=32, chunks=conv_chunks)

    (y2,) = _tc_mid(S1, y1, b1, n1_g, n1_b, W2, dinv, 64, 128)
    (S2,) = _sc_scatter_rows((y2.reshape(2 * _NP, 64),), src2, dstg,
                             zeros[64], parts=2, c2p=64, chunks=conv_chunks)

    y3a, y3b = _tc_mid(S2, y2, b2, n2_g, n2_b, W3, dinv, 128, 256)
    S3a, S3b = _sc_scatter_rows(
        (y3a.reshape(2 * _NP, 64), y3b.reshape(2 * _NP, 64)), src2, dstg,
        zeros[64], parts=4, c2p=64, chunks=conv_chunks)

    return _tc_final(S3a, S3b, y3a, y3b, b3, n3_g, n3_b, dinv, batch_w,
                     Wo, bo)


# speedup vs baseline: 27.3921x; 1.0055x over previous
import functools

import jax
import jax.numpy as jnp
from jax import lax
from jax.experimental import pallas as pl
from jax.experimental.pallas import tpu as pltpu
from jax.experimental.pallas import tpu_sc as plsc

_N = 10000
_E = 320000
_NG = 16
_TILES = 16
_CORES = 2
_K = 125
_NP = 10240
_RPT = _NP // _TILES
_BLK = 1024
_GRID = _NP // _BLK

_F32 = jnp.float32
_HI = lax.Precision.HIGHEST


def _dot(a, b, dims=None):
    dn = (((a.ndim - 1,), (0,)), ((), ())) if dims is None else dims
    return lax.dot_general(a, b, dn, precision=_HI,
                           preferred_element_type=_F32)


def _ln(t, g, b, eps=1e-5):
    mu = jnp.mean(t, axis=-1, keepdims=True)
    var = jnp.mean((t - mu) ** 2, axis=-1, keepdims=True)
    return (t - mu) * lax.rsqrt(var + eps) * g + b


def _sc_mesh():
    return plsc.VectorSubcoreMesh(core_axis_name="c", subcore_axis_name="s")


@functools.partial(jax.jit, static_argnames=("parts", "c2p", "chunks"))
def _sc_scatter_rows(ytabs, srcp, dstg, zeros, *, parts, c2p, chunks):
    ppc = parts // _CORES
    gpt = 128 // c2p
    ntab = max(1, parts // gpt)
    nb = 4
    dd = 2
    groups = chunks // nb + 1

    @functools.partial(
        pl.kernel,
        mesh=_sc_mesh(),
        compiler_params=pltpu.CompilerParams(use_tc_tiling_on_sc=False),
        out_type=[jax.ShapeDtypeStruct((_NP, 128), _F32)] * ntab,
        scratch_types=[
            pltpu.VMEM((chunks, _K), jnp.int32),
            pltpu.VMEM((chunks, _K), jnp.int32),
        ]
        + [pltpu.VMEM((_K, c2p), _F32) for _ in range(nb)]
        + [pltpu.SemaphoreType.DMA for _ in range(2 * nb)]
        + [pltpu.VMEM_SHARED((_NP, c2p), _F32)],
    )
    def k(*refs):
        tabs = refs[:ntab]
        srcp0_hbm, srcp1_hbm, dst_hbm, z_hbm = refs[ntab:ntab + 4]
        outs = refs[ntab + 4:2 * ntab + 4]
        rest = refs[2 * ntab + 4:]
        src_v, dst_v = rest[0], rest[1]
        gbufs = rest[2:2 + nb]
        gsems = rest[2 + nb:2 + 2 * nb]
        ssems = rest[2 + 2 * nb:2 + 3 * nb]
        acc_sh = rest[2 + 3 * nb]
        c = lax.axis_index("c")
        s = lax.axis_index("s")
        pltpu.sync_copy(dst_hbm.at[s], dst_v)
        r0 = s * _RPT

        def gather_wait(b):
            pltpu.make_async_copy(tabs[0].at[src_v.at[0]], gbufs[b],
                                  gsems[b]).wait()

        def scatter_start(j, b):
            pltpu.async_copy(gbufs[b], acc_sh.at[dst_v.at[j]], ssems[b],
                             add=True)

        def scatter_wait(b):
            pltpu.make_async_copy(gbufs[b], acc_sh.at[dst_v.at[0]],
                                  ssems[b]).wait()

        srcps = (srcp0_hbm, srcp1_hbm)
        for p in range(ppc):
            par = [(cc * ppc + p) % gpt for cc in range(_CORES)]
            tix = [(cc * ppc + p) // gpt for cc in range(_CORES)]

            if par[0] == par[1]:
                pltpu.sync_copy(srcps[par[0]].at[s], src_v)
            else:
                pl.when(c == 0)(
                    lambda: pltpu.sync_copy(srcps[par[0]].at[s], src_v))
                pl.when(c == 1)(
                    lambda: pltpu.sync_copy(srcps[par[1]].at[s], src_v))

            def gather_start(j, b, tix=tix):
                if tix[0] == tix[1]:
                    pltpu.async_copy(tabs[tix[0]].at[src_v.at[j]], gbufs[b],
                                     gsems[b])
                else:
                    def g0():
                        pltpu.async_copy(tabs[tix[0]].at[src_v.at[j]],
                                         gbufs[b], gsems[b])

                    def g1():
                        pltpu.async_copy(tabs[tix[1]].at[src_v.at[j]],
                                         gbufs[b], gsems[b])

                    pl.when(c == 0)(g0)
                    pl.when(c == 1)(g1)

            pltpu.sync_copy(z_hbm.at[pl.ds(r0, _RPT)],
                            acc_sh.at[pl.ds(r0, _RPT)])
            plsc.subcore_barrier()

            def body(g, carry):
                for b in range(nb):
                    j = g * nb + b
                    pl.when(g >= 1)(lambda b=b: scatter_wait(b))
                    pl.when(g < groups - 1)(lambda j=j, b=b: gather_start(j, b))
                    b2 = (b - dd) % nb
                    jd = j - dd

                    def drain(jd=jd, b2=b2):
                        gather_wait(b2)
                        scatter_start(jd, b2)

                    if b < dd:
                        pl.when(g >= 1)(drain)
                    else:
                        pl.when(g < groups - 1)(drain)
                return carry

            lax.fori_loop(0, groups, body, 0)
            plsc.subcore_barrier()

            def write_for(part):
                out = outs[part // gpt]
                coff = (part % gpt) * c2p

                def go():
                    pltpu.sync_copy(
                        acc_sh.at[pl.ds(r0, _RPT)],
                        out.at[pl.ds(r0, _RPT), pl.ds(coff, c2p)])
                return go

            pl.when(c == 0)(write_for(0 * ppc + p))
            pl.when(c == 1)(write_for(1 * ppc + p))
            plsc.subcore_barrier()

    return k(*ytabs, srcp[0], srcp[1], dstg, zeros)


@jax.jit
def _sc_degree(dstd, ones8, zeros8):
    chunks = dstd.shape[2]

    @functools.partial(
        pl.kernel,
        mesh=_sc_mesh(),
        compiler_params=pltpu.CompilerParams(use_tc_tiling_on_sc=False),
        out_type=jax.ShapeDtypeStruct((_NP, 128), _F32),
        scratch_types=[
            pltpu.VMEM((chunks, _K), jnp.int32),
            pltpu.VMEM((_K, 8), _F32),
            pltpu.VMEM_SHARED((_NP, 8), _F32),
            pltpu.SemaphoreType.DMA,
        ],
    )
    def k(dst_hbm, ones_hbm, z_hbm, out_hbm, dst_v, obuf, acc_sh, sem):
        c = lax.axis_index("c")
        s = lax.axis_index("s")
        pltpu.sync_copy(dst_hbm.at[c, s], dst_v)
        pltpu.sync_copy(ones_hbm, obuf)
        r0 = s * _RPT
        pltpu.sync_copy(z_hbm.at[pl.ds(r0, _RPT)], acc_sh.at[pl.ds(r0, _RPT)])
        plsc.subcore_barrier()

        def body(g, carry):
            for b in range(4):
                pltpu.async_copy(obuf, acc_sh.at[dst_v.at[g * 4 + b]], sem,
                                 add=True)
            for b in range(4):
                pltpu.make_async_copy(obuf, acc_sh.at[dst_v.at[0]],
                                      sem).wait()
            return carry

        lax.fori_loop(0, chunks // 4, body, 0)
        plsc.subcore_barrier()

        def w0():
            pltpu.sync_copy(acc_sh.at[pl.ds(r0, _RPT)],
                            out_hbm.at[pl.ds(r0, _RPT), pl.ds(0, 8)])

        def w1():
            pltpu.sync_copy(acc_sh.at[pl.ds(r0, _RPT)],
                            out_hbm.at[pl.ds(r0, _RPT), pl.ds(8, 8)])

        pl.when(c == 0)(w0)
        pl.when(c == 1)(w1)

    return k(dstd, ones8, zeros8)


def _row_spec():
    return pl.BlockSpec((_BLK, 128), lambda i: (i, 0))


def _full(shape):
    nd = len(shape)
    return pl.BlockSpec(shape, lambda i, _nd=nd: (0,) * _nd)


def _k1_body(x_ref, win_ref, bin_ref, g_ref, b_ref, w1_ref, deg_ref,
             y_ref, dv_ref):
    dinv = lax.rsqrt(deg_ref[:, 0:1] + deg_ref[:, 8:9] + 1.0)
    dv_ref[...] = jnp.broadcast_to(dinv, (_BLK, 128))
    h = jax.nn.relu(_ln(_dot(x_ref[...], win_ref[...]) + bin_ref[...],
                        g_ref[...], b_ref[...]))
    y = _dot(h, w1_ref[...]) * dinv
    y_ref[...] = jnp.concatenate([y, jnp.zeros_like(y)], axis=1)


def _mid2_body(s_ref, y_ref, bias_ref, g_ref, b_ref, w_ref, dv_ref, *yout,
               cin=64):
    dinv = dv_ref[:, 0:1]
    t = (s_ref[:, :cin] + y_ref[:, :cin]) * dinv + bias_ref[...]
    h = jax.nn.relu(_ln(t, g_ref[...], b_ref[...]))
    y = _dot(h, w_ref[...]) * dinv
    for q, r in enumerate(yout):
        r[...] = y[:, q * 128:(q + 1) * 128]


def _fin_body(sa_ref, sb_ref, ya_ref, yb_ref, bias_ref, g_ref, b_ref, dv_ref,
              bat_ref, wo_ref, bo_ref, o_ref, sums_ref, cnt_ref):
    i = pl.program_id(0)

    @pl.when(i == 0)
    def _():
        sums_ref[...] = jnp.zeros_like(sums_ref)
        cnt_ref[...] = jnp.zeros_like(cnt_ref)

    dinv = dv_ref[:, 0:1]
    t = jnp.concatenate([sa_ref[...] + ya_ref[...],
                         sb_ref[...] + yb_ref[...]], axis=1)
    t = t * dinv + bias_ref[...]
    h = jax.nn.relu(_ln(t, g_ref[...], b_ref[...]))
    iota = lax.broadcasted_iota(jnp.int32, (_BLK, _NG), 1).astype(_F32)
    oht = (bat_ref[:, 0:1] == iota).astype(_F32)
    sums_ref[...] += _dot(oht, h, dims=(((0,), (0,)), ((), ())))
    cnt_ref[...] += jnp.sum(oht, axis=0, keepdims=True)

    @pl.when(i == _GRID - 1)
    def _():
        recip = 1.0 / jnp.maximum(cnt_ref[...], 1.0)
        eye = (lax.broadcasted_iota(jnp.int32, (_NG, _NG), 0)
               == lax.broadcasted_iota(jnp.int32, (_NG, _NG), 1)).astype(_F32)
        pooled = _dot(eye * recip, sums_ref[...])
        o_ref[...] = _dot(pooled, wo_ref[...]) + bo_ref[...]


def _tc_stage1(x, W_in, b_in, g, b, W1, deg):
    return pl.pallas_call(
        _k1_body,
        grid=(_GRID,),
        in_specs=[_row_spec(), _full((128, 32)), _full((32,)), _full((32,)),
                  _full((32,)), _full((32, 64)), _row_spec()],
        out_specs=[_row_spec()] * 2,
        out_shape=[jax.ShapeDtypeStruct((_NP, 128), _F32)] * 2,
    )(x, W_in, b_in, g, b, W1, deg)


def _tc_mid(S, y, bias, g, b, W, dinv, cin, cout):
    body = functools.partial(_mid2_body, cin=cin)
    nout = cout // 128
    return pl.pallas_call(
        body,
        grid=(_GRID,),
        in_specs=[_row_spec(), _row_spec(), _full((cin,)), _full((cin,)),
                  _full((cin,)), _full((cin, cout)), _row_spec()],
        out_specs=[_row_spec()] * nout,
        out_shape=[jax.ShapeDtypeStruct((_NP, 128), _F32)] * nout,
    )(S, y, bias, g, b, W, dinv)


def _tc_final(Sa, Sb, ya, yb, bias, g, b, dinv, batch_f, Wo, bo):
    return pl.pallas_call(
        _fin_body,
        grid=(_GRID,),
        in_specs=[_row_spec(), _row_spec(), _row_spec(), _row_spec(),
                  _full((256,)), _full((256,)), _full((256,)), _row_spec(),
                  _row_spec(), _full((256, 64)), _full((64,))],
        out_specs=pl.BlockSpec((_NG, 64), lambda i: (0, 0)),
        out_shape=jax.ShapeDtypeStruct((_NG, 64), _F32),
        scratch_shapes=[pltpu.VMEM((_NG, 256), _F32),
                        pltpu.VMEM((1, _NG), _F32)],
    )(Sa, Sb, ya, yb, bias, g, b, dinv, batch_f, Wo, bo)


def kernel(x, edge_index, batch, W_in, b_in, ln_in_g, ln_in_b, W1, b1, n1_g,
           n1_b, W2, b2, n2_g, n2_b, W3, b3, n3_g, n3_b, Wo, bo):
    src = edge_index[0].astype(jnp.int32)
    dst = edge_index[1].astype(jnp.int32)

    conv_chunks = _E // (_TILES * _K)
    deg_chunks = _E // (_CORES * _TILES * _K)

    def idx(mult, q):
        return (mult * src + q).reshape(_TILES, conv_chunks, _K)

    src2 = (idx(2, 0), idx(2, 1))
    src4 = (idx(4, 0), idx(4, 1))
    dstg = dst.reshape(_TILES, conv_chunks, _K)
    dstd = dst.reshape(_CORES, _TILES, deg_chunks, _K)

    z8 = jnp.zeros((_NP, 8), _F32)
    ones8 = jnp.ones((_K, 8), _F32)
    zeros = {c2: jnp.zeros((_NP, c2), _F32) for c2 in (32, 64)}
    x_p = jnp.pad(x, ((0, _NP - _N), (0, 0)))
    batch_f = jnp.pad(batch.astype(_F32), (0, _NP - _N),
                      constant_values=-1.0).reshape(_NP, 1)
    batch_w = jnp.broadcast_to(batch_f, (_NP, 128))

    deg = _sc_degree(dstd, ones8, z8)

    y1, dinv = _tc_stage1(x_p, W_in, b_in, ln_in_g, ln_in_b, W1, deg)
    (S1,) = _sc_scatter_rows((y1.reshape(4 * _NP, 32),), src4, dstg,
                             zeros[32], parts=2, c2p=32, chunks=conv_chunks)

    (y2,) = _tc_mid(S1, y1, b1, n1_g, n1_b, W2, dinv, 64, 128)
    (S2,) = _sc_scatter_rows((y2.reshape(2 * _NP, 64),), src2, dstg,
                             zeros[64], parts=2, c2p=64, chunks=conv_chunks)

    y3a, y3b = _tc_mid(S2, y2, b2, n2_g, n2_b, W3, dinv, 128, 256)
    S3a, S3b = _sc_scatter_rows(
        (y3a.reshape(2 * _NP, 64), y3b.reshape(2 * _NP, 64)), src2, dstg,
        zeros[64], parts=4, c2p=64, chunks=conv_chunks)

    return _tc_final(S3a, S3b, y3a, y3b, b3, n3_g, n3_b, dinv, batch_w,
                     Wo, bo)
